# baseline (speedup 1.0000x reference)
import math

import jax
import jax.numpy as jnp
import numpy as np
from jax import lax
from jax.experimental import pallas as pl
from jax.experimental.pallas import tpu as pltpu
from jax.experimental.pallas import tpu_sc as plsc

NUM_VOCAB = 1000000
EMBED_DIM = 64
MAXLEN = 512
BATCH = 1024
SEQLEN = 200

NC = 2
NS = 16
NW = NC * NS

SWATH = 128
NSWATH = BATCH // SWATH
NRANGE = NW // NSWATH
SBLK = 48
SMAX = SEQLEN - (NRANGE - 1) * SBLK
VEC = 16
GPITCH = 2 * EMBED_DIM + 1


def _make_pe(maxlen, d_model):
    position = np.arange(maxlen, dtype=np.float32)[:, None]
    div_term = np.exp(
        np.arange(0, d_model, 2).astype(np.float32) * (-math.log(10000.0) / d_model)
    )
    pe = np.zeros((maxlen, d_model), dtype=np.float32)
    pe[:, 0::2] = np.sin(position * div_term)
    pe[:, 1::2] = np.cos(position * div_term)
    return pe


_PE_FLAT = _make_pe(MAXLEN, EMBED_DIM)[:SEQLEN].reshape(SEQLEN * EMBED_DIM)


def _sc_embed(tokens_t, table_pairs, pe):
    mesh = plsc.VectorSubcoreMesh(core_axis_name="c", subcore_axis_name="s")

    @pl.kernel(
        out_type=jax.ShapeDtypeStruct((SEQLEN, EMBED_DIM, BATCH), jnp.float32),
        mesh=mesh,
        compiler_params=pltpu.CompilerParams(needs_layout_passes=False),
        scratch_types=[
            pltpu.VMEM((SMAX, SWATH), jnp.int32),
            pltpu.VMEM((SMAX * EMBED_DIM,), jnp.float32),
            pltpu.VMEM((SWATH,), jnp.int32),
            pltpu.VMEM((SWATH,), jnp.int32),
            pltpu.VMEM((SWATH, GPITCH), jnp.float32),
            pltpu.VMEM((SWATH, GPITCH), jnp.float32),
            pltpu.VMEM((EMBED_DIM, SWATH), jnp.float32),
            pltpu.VMEM((EMBED_DIM, SWATH), jnp.float32),
            pltpu.SemaphoreType.DMA,
            pltpu.SemaphoreType.DMA,
            pltpu.SemaphoreType.DMA,
            pltpu.SemaphoreType.DMA,
        ],
    )
    def k(tok_hbm, tab_hbm, pe_hbm, out_hbm,
          idx_v, pe_v, qa, qb, ga, gb, ta, tb, sga, sgb, soa, sob):
        wid = lax.axis_index("s") * NC + lax.axis_index("c")
        j = lax.rem(wid, NSWATH)
        r = wid // NSWATH
        s0 = pl.multiple_of(r * SBLK, 8)
        jb = pl.multiple_of(j * SWATH, SWATH)
        ns = lax.select(r == NRANGE - 1, SMAX, SBLK)

        pltpu.make_async_copy(
            tok_hbm.at[pl.ds(s0, SMAX), pl.ds(jb, SWATH)], idx_v, sga
        ).start()
        pltpu.make_async_copy(
            pe_hbm.at[pl.ds(pl.multiple_of(s0 * EMBED_DIM, 1024),
                            SMAX * EMBED_DIM)], pe_v, sgb
        ).start()
        pltpu.make_async_copy(
            tok_hbm.at[pl.ds(s0, SMAX), pl.ds(jb, SWATH)], idx_v, sga
        ).wait()
        pltpu.make_async_copy(
            pe_hbm.at[pl.ds(pl.multiple_of(s0 * EMBED_DIM, 1024),
                            SMAX * EMBED_DIM)], pe_v, sgb
        ).wait()

        iota = lax.iota(jnp.int32, VEC)
        dcol = [(d * VEC + iota) for d in range(EMBED_DIM // VEC)]

        def make_q(s_local, q):
            for m in range(SWATH // VEC):
                sl = pl.ds(m * VEC, VEC)
                q[sl] = lax.shift_right_logical(idx_v[s_local, sl], 1)

        def gather_start(q, g, sem):
            pltpu.make_async_copy(
                tab_hbm.at[q], g.at[:, pl.ds(0, 2 * EMBED_DIM)], sem).start()

        def gather_wait(q, g, sem):
            pltpu.make_async_copy(
                tab_hbm.at[q], g.at[:, pl.ds(0, 2 * EMBED_DIM)], sem).wait()

        rows = [m * VEC + iota for m in range(SWATH // VEC)]

        def compute(s_local, g, t):
            h64s = [lax.shift_left(idx_v[s_local, pl.ds(m * VEC, VEC)] & 1, 6)
                    for m in range(SWATH // VEC)]

            def d_body(d, carry):
                pe_spl = plsc.load_gather(
                    pe_v, [jnp.full((VEC,), s_local * EMBED_DIM + d, jnp.int32)])
                for m in range(SWATH // VEC):
                    col = h64s[m] + d
                    v = plsc.load_gather(g, [rows[m], col])
                    t[d, pl.ds(m * VEC, VEC)] = v * 8.0 + pe_spl
                return carry

            lax.fori_loop(0, EMBED_DIM, d_body, 0)

        def out_start(s_local, t, sem):
            pltpu.make_async_copy(
                t, out_hbm.at[s0 + s_local, :, pl.ds(jb, SWATH)], sem
            ).start()

        def out_wait(s_local, t, sem):
            pltpu.make_async_copy(
                t, out_hbm.at[s0 + s_local, :, pl.ds(jb, SWATH)], sem
            ).wait()

        make_q(0, qa)
        gather_start(qa, ga, sga)

        def pair_body(ss, carry):
            s_a = 2 * ss
            s_b = 2 * ss + 1
            make_q(s_b, qb)
            gather_start(qb, gb, sgb)
            gather_wait(qa, ga, sga)

            @pl.when(ss > 0)
            def _():
                out_wait(s_a - 2, ta, soa)

            compute(s_a, ga, ta)
            out_start(s_a, ta, soa)

            @pl.when(s_b + 1 < ns)
            def _():
                make_q(s_b + 1, qa)
                gather_start(qa, ga, sga)

            gather_wait(qb, gb, sgb)

            @pl.when(ss > 0)
            def _():
                out_wait(s_b - 2, tb, sob)

            compute(s_b, gb, tb)
            out_start(s_b, tb, sob)
            return carry

        lax.fori_loop(0, ns // 2, pair_body, 0)
        out_wait(ns - 2, ta, soa)
        out_wait(ns - 1, tb, sob)

    return k(tokens_t, table_pairs, pe)


def kernel(tokens, table):
    tokens_t = tokens.T.astype(jnp.int32)
    table_pairs = jnp.reshape(
        jnp.transpose(jnp.reshape(table.T, (EMBED_DIM, NUM_VOCAB // 2, 2)),
                      (1, 2, 0)),
        (NUM_VOCAB // 2, 2 * EMBED_DIM),
    )
    out = _sc_embed(tokens_t, table_pairs, jnp.asarray(_PE_FLAT))
    return jnp.transpose(out, (2, 0, 1))


# speedup vs baseline: 1.2295x; 1.2295x over previous
import math

import jax
import jax.numpy as jnp
import numpy as np
from jax import lax
from jax.experimental import pallas as pl
from jax.experimental.pallas import tpu as pltpu
from jax.experimental.pallas import tpu_sc as plsc

NUM_VOCAB = 1000000
EMBED_DIM = 64
MAXLEN = 512
BATCH = 1024
SEQLEN = 200

NC = 2
NS = 16
NW = NC * NS

SWATH = 128
NSWATH = BATCH // SWATH
NRANGE = NW // NSWATH
SBLK = 48
SMAX = SEQLEN - (NRANGE - 1) * SBLK
VEC = 16
GPITCH = 2 * EMBED_DIM + 1


def _make_pe(maxlen, d_model):
    position = np.arange(maxlen, dtype=np.float32)[:, None]
    div_term = np.exp(
        np.arange(0, d_model, 2).astype(np.float32) * (-math.log(10000.0) / d_model)
    )
    pe = np.zeros((maxlen, d_model), dtype=np.float32)
    pe[:, 0::2] = np.sin(position * div_term)
    pe[:, 1::2] = np.cos(position * div_term)
    return pe


_PE_FLAT = _make_pe(MAXLEN, EMBED_DIM)[:SEQLEN].reshape(SEQLEN * EMBED_DIM)


def _sc_embed(tokens_t, table_pairs, pe):
    mesh = plsc.VectorSubcoreMesh(core_axis_name="c", subcore_axis_name="s")

    @pl.kernel(
        out_type=jax.ShapeDtypeStruct((SEQLEN, EMBED_DIM, BATCH), jnp.float32),
        mesh=mesh,
        compiler_params=pltpu.CompilerParams(needs_layout_passes=False),
        scratch_types=[
            pltpu.VMEM((SMAX, SWATH), jnp.int32),
            pltpu.VMEM((SMAX * EMBED_DIM,), jnp.float32),
            pltpu.VMEM((SWATH,), jnp.int32),
            pltpu.VMEM((SWATH,), jnp.int32),
            pltpu.VMEM((SWATH, GPITCH), jnp.float32),
            pltpu.VMEM((SWATH, GPITCH), jnp.float32),
            pltpu.VMEM((EMBED_DIM, SWATH), jnp.float32),
            pltpu.VMEM((EMBED_DIM, SWATH), jnp.float32),
            pltpu.SemaphoreType.DMA,
            pltpu.SemaphoreType.DMA,
            pltpu.SemaphoreType.DMA,
            pltpu.SemaphoreType.DMA,
        ],
    )
    def k(tok_hbm, tab_hbm, pe_hbm, out_hbm,
          idx_v, pe_v, qa, qb, ga, gb, ta, tb, sga, sgb, soa, sob):
        wid = lax.axis_index("s") * NC + lax.axis_index("c")
        j = lax.rem(wid, NSWATH)
        r = wid // NSWATH
        s0 = pl.multiple_of(r * SBLK, 8)
        jb = pl.multiple_of(j * SWATH, SWATH)
        ns = lax.select(r == NRANGE - 1, SMAX, SBLK)

        pltpu.make_async_copy(
            tok_hbm.at[pl.ds(s0, SMAX), pl.ds(jb, SWATH)], idx_v, sga
        ).start()
        pltpu.make_async_copy(
            pe_hbm.at[pl.ds(pl.multiple_of(s0 * EMBED_DIM, 1024),
                            SMAX * EMBED_DIM)], pe_v, sgb
        ).start()
        pltpu.make_async_copy(
            tok_hbm.at[pl.ds(s0, SMAX), pl.ds(jb, SWATH)], idx_v, sga
        ).wait()
        pltpu.make_async_copy(
            pe_hbm.at[pl.ds(pl.multiple_of(s0 * EMBED_DIM, 1024),
                            SMAX * EMBED_DIM)], pe_v, sgb
        ).wait()

        iota = lax.iota(jnp.int32, VEC)
        dcol = [(d * VEC + iota) for d in range(EMBED_DIM // VEC)]

        def make_q(s_local, q):
            for m in range(SWATH // VEC):
                sl = pl.ds(m * VEC, VEC)
                q[sl] = lax.shift_right_logical(idx_v[s_local, sl], 1)

        def gather_start(q, g, sem):
            pltpu.make_async_copy(
                tab_hbm.at[q], g.at[:, pl.ds(0, 2 * EMBED_DIM)], sem).start()

        def gather_wait(q, g, sem):
            pltpu.make_async_copy(
                tab_hbm.at[q], g.at[:, pl.ds(0, 2 * EMBED_DIM)], sem).wait()

        rows = [m * VEC + iota for m in range(SWATH // VEC)]

        def compute(s_local, g, t):
            h64s = [lax.shift_left(idx_v[s_local, pl.ds(m * VEC, VEC)] & 1, 6)
                    for m in range(SWATH // VEC)]

            @plsc.parallel_loop(0, EMBED_DIM, unroll=4)
            def d_body(d):
                pe_spl = plsc.load_gather(
                    pe_v, [jnp.full((VEC,), s_local * EMBED_DIM + d, jnp.int32)])
                for m in range(SWATH // VEC):
                    col = h64s[m] + d
                    v = plsc.load_gather(g, [rows[m], col])
                    t[d, pl.ds(m * VEC, VEC)] = v * 8.0 + pe_spl

        def out_start(s_local, t, sem):
            pltpu.make_async_copy(
                t, out_hbm.at[s0 + s_local, :, pl.ds(jb, SWATH)], sem
            ).start()

        def out_wait(s_local, t, sem):
            pltpu.make_async_copy(
                t, out_hbm.at[s0 + s_local, :, pl.ds(jb, SWATH)], sem
            ).wait()

        make_q(0, qa)
        gather_start(qa, ga, sga)

        def pair_body(ss, carry):
            s_a = 2 * ss
            s_b = 2 * ss + 1
            make_q(s_b, qb)
            gather_start(qb, gb, sgb)
            gather_wait(qa, ga, sga)

            @pl.when(ss > 0)
            def _():
                out_wait(s_a - 2, ta, soa)

            compute(s_a, ga, ta)
            out_start(s_a, ta, soa)

            @pl.when(s_b + 1 < ns)
            def _():
                make_q(s_b + 1, qa)
                gather_start(qa, ga, sga)

            gather_wait(qb, gb, sgb)

            @pl.when(ss > 0)
            def _():
                out_wait(s_b - 2, tb, sob)

            compute(s_b, gb, tb)
            out_start(s_b, tb, sob)
            return carry

        lax.fori_loop(0, ns // 2, pair_body, 0)
        out_wait(ns - 2, ta, soa)
        out_wait(ns - 1, tb, sob)

    return k(tokens_t, table_pairs, pe)


def kernel(tokens, table):
    tokens_t = tokens.T.astype(jnp.int32)
    table_pairs = jnp.reshape(
        jnp.transpose(jnp.reshape(table.T, (EMBED_DIM, NUM_VOCAB // 2, 2)),
                      (1, 2, 0)),
        (NUM_VOCAB // 2, 2 * EMBED_DIM),
    )
    out = _sc_embed(tokens_t, table_pairs, jnp.asarray(_PE_FLAT))
    return jnp.transpose(out, (2, 0, 1))


# speedup vs baseline: 1.4625x; 1.1895x over previous
import math

import jax
import jax.numpy as jnp
import numpy as np
from jax import lax
from jax.experimental import pallas as pl
from jax.experimental.pallas import tpu as pltpu
from jax.experimental.pallas import tpu_sc as plsc

NUM_VOCAB = 1000000
EMBED_DIM = 64
MAXLEN = 512
BATCH = 1024
SEQLEN = 200

NC = 2
NS = 16
NW = NC * NS

SWATH = 128
NSWATH = BATCH // SWATH
NRANGE = NW // NSWATH
SBLK = 48
SMAX = SEQLEN - (NRANGE - 1) * SBLK
VEC = 16
GPITCH = 2 * EMBED_DIM + 1


def _make_pe(maxlen, d_model):
    position = np.arange(maxlen, dtype=np.float32)[:, None]
    div_term = np.exp(
        np.arange(0, d_model, 2).astype(np.float32) * (-math.log(10000.0) / d_model)
    )
    pe = np.zeros((maxlen, d_model), dtype=np.float32)
    pe[:, 0::2] = np.sin(position * div_term)
    pe[:, 1::2] = np.cos(position * div_term)
    return pe


_PE_FLAT = _make_pe(MAXLEN, EMBED_DIM)[:SEQLEN].reshape(SEQLEN * EMBED_DIM)


def _sc_embed(tokens_t, table_pairs, pe):
    mesh = plsc.VectorSubcoreMesh(core_axis_name="c", subcore_axis_name="s")

    @pl.kernel(
        out_type=jax.ShapeDtypeStruct((SEQLEN, EMBED_DIM, BATCH), jnp.float32),
        mesh=mesh,
        compiler_params=pltpu.CompilerParams(needs_layout_passes=False),
        scratch_types=[
            pltpu.VMEM((SMAX, SWATH), jnp.int32),
            pltpu.VMEM((SMAX * EMBED_DIM,), jnp.float32),
            pltpu.VMEM((SWATH,), jnp.int32),
            pltpu.VMEM((SWATH,), jnp.int32),
            pltpu.VMEM((SWATH, GPITCH), jnp.float32),
            pltpu.VMEM((SWATH, GPITCH), jnp.float32),
            pltpu.VMEM((EMBED_DIM, SWATH), jnp.float32),
            pltpu.VMEM((EMBED_DIM, SWATH), jnp.float32),
            pltpu.SemaphoreType.DMA,
            pltpu.SemaphoreType.DMA,
            pltpu.SemaphoreType.DMA,
            pltpu.SemaphoreType.DMA,
        ],
    )
    def k(tok_hbm, tab_hbm, pe_hbm, out_hbm,
          idx_v, pe_v, qa, qb, ga, gb, ta, tb, sga, sgb, soa, sob):
        wid = lax.axis_index("s") * NC + lax.axis_index("c")
        j = lax.rem(wid, NSWATH)
        r = wid // NSWATH
        s0 = pl.multiple_of(r * SBLK, 8)
        jb = pl.multiple_of(j * SWATH, SWATH)
        ns = lax.select(r == NRANGE - 1, SMAX, SBLK)

        pltpu.make_async_copy(
            tok_hbm.at[pl.ds(s0, SMAX), pl.ds(jb, SWATH)], idx_v, sga
        ).start()
        pltpu.make_async_copy(
            pe_hbm.at[pl.ds(pl.multiple_of(s0 * EMBED_DIM, 1024),
                            SMAX * EMBED_DIM)], pe_v, sgb
        ).start()
        pltpu.make_async_copy(
            tok_hbm.at[pl.ds(s0, SMAX), pl.ds(jb, SWATH)], idx_v, sga
        ).wait()
        pltpu.make_async_copy(
            pe_hbm.at[pl.ds(pl.multiple_of(s0 * EMBED_DIM, 1024),
                            SMAX * EMBED_DIM)], pe_v, sgb
        ).wait()

        iota = lax.iota(jnp.int32, VEC)
        dcol = [(d * VEC + iota) for d in range(EMBED_DIM // VEC)]

        def make_q(s_local, q):
            for m in range(SWATH // VEC):
                sl = pl.ds(m * VEC, VEC)
                q[sl] = lax.shift_right_logical(idx_v[s_local, sl], 1)

        def gather_start(q, g, sem):
            pltpu.make_async_copy(
                tab_hbm.at[q], g.at[:, pl.ds(0, 2 * EMBED_DIM)], sem).start()

        def gather_wait(q, g, sem):
            pltpu.make_async_copy(
                tab_hbm.at[q], g.at[:, pl.ds(0, 2 * EMBED_DIM)], sem).wait()

        rows = [m * VEC + iota for m in range(SWATH // VEC)]

        def compute(s_local, g, t):
            h64s = [lax.shift_left(idx_v[s_local, pl.ds(m * VEC, VEC)] & 1, 6)
                    for m in range(SWATH // VEC)]

            @plsc.parallel_loop(0, EMBED_DIM, unroll=8)
            def d_body(d):
                pe_spl = plsc.load_gather(
                    pe_v, [jnp.full((VEC,), s_local * EMBED_DIM + d, jnp.int32)])
                for m in range(SWATH // VEC):
                    col = h64s[m] + d
                    v = plsc.load_gather(g, [rows[m], col])
                    t[d, pl.ds(m * VEC, VEC)] = v * 8.0 + pe_spl

        def out_start(s_local, t, sem):
            pltpu.make_async_copy(
                t, out_hbm.at[s0 + s_local, :, pl.ds(jb, SWATH)], sem
            ).start()

        def out_wait(s_local, t, sem):
            pltpu.make_async_copy(
                t, out_hbm.at[s0 + s_local, :, pl.ds(jb, SWATH)], sem
            ).wait()

        make_q(0, qa)
        gather_start(qa, ga, sga)

        def pair_body(ss, carry):
            s_a = 2 * ss
            s_b = 2 * ss + 1
            make_q(s_b, qb)
            gather_start(qb, gb, sgb)
            gather_wait(qa, ga, sga)

            @pl.when(ss > 0)
            def _():
                out_wait(s_a - 2, ta, soa)

            compute(s_a, ga, ta)
            out_start(s_a, ta, soa)

            @pl.when(s_b + 1 < ns)
            def _():
                make_q(s_b + 1, qa)
                gather_start(qa, ga, sga)

            gather_wait(qb, gb, sgb)

            @pl.when(ss > 0)
            def _():
                out_wait(s_b - 2, tb, sob)

            compute(s_b, gb, tb)
            out_start(s_b, tb, sob)
            return carry

        lax.fori_loop(0, ns // 2, pair_body, 0)
        out_wait(ns - 2, ta, soa)
        out_wait(ns - 1, tb, sob)

    return k(tokens_t, table_pairs, pe)


def kernel(tokens, table):
    tokens_t = tokens.T.astype(jnp.int32)
    table_pairs = table.reshape(NUM_VOCAB // 2, 2 * EMBED_DIM)
    out = _sc_embed(tokens_t, table_pairs, jnp.asarray(_PE_FLAT))
    return jnp.transpose(out, (2, 0, 1))


# speedup vs baseline: 1.4695x; 1.0047x over previous
import math

import jax
import jax.numpy as jnp
import numpy as np
from jax import lax
from jax.experimental import pallas as pl
from jax.experimental.pallas import tpu as pltpu
from jax.experimental.pallas import tpu_sc as plsc

NUM_VOCAB = 1000000
EMBED_DIM = 64
MAXLEN = 512
BATCH = 1024
SEQLEN = 200

NC = 2
NS = 16
NW = NC * NS

SWATH = 128
NSWATH = BATCH // SWATH
NRANGE = NW // NSWATH
SBLK = 48
SMAX = SEQLEN - (NRANGE - 1) * SBLK
VEC = 16
GPITCH = 2 * EMBED_DIM + 1


def _make_pe(maxlen, d_model):
    position = np.arange(maxlen, dtype=np.float32)[:, None]
    div_term = np.exp(
        np.arange(0, d_model, 2).astype(np.float32) * (-math.log(10000.0) / d_model)
    )
    pe = np.zeros((maxlen, d_model), dtype=np.float32)
    pe[:, 0::2] = np.sin(position * div_term)
    pe[:, 1::2] = np.cos(position * div_term)
    return pe


_PE_FLAT = _make_pe(MAXLEN, EMBED_DIM)[:SEQLEN].reshape(SEQLEN * EMBED_DIM)


def _sc_embed(tokens_t, table_pairs, pe):
    mesh = plsc.VectorSubcoreMesh(core_axis_name="c", subcore_axis_name="s")

    @pl.kernel(
        out_type=jax.ShapeDtypeStruct((SEQLEN, EMBED_DIM, BATCH), jnp.float32),
        mesh=mesh,
        compiler_params=pltpu.CompilerParams(needs_layout_passes=False),
        scratch_types=[
            pltpu.VMEM((SMAX, SWATH), jnp.int32),
            pltpu.VMEM((SMAX * EMBED_DIM,), jnp.float32),
            pltpu.VMEM((SWATH,), jnp.int32),
            pltpu.VMEM((SWATH,), jnp.int32),
            pltpu.VMEM((SWATH, GPITCH), jnp.float32),
            pltpu.VMEM((SWATH, GPITCH), jnp.float32),
            pltpu.VMEM((EMBED_DIM, SWATH), jnp.float32),
            pltpu.VMEM((EMBED_DIM, SWATH), jnp.float32),
            pltpu.SemaphoreType.DMA,
            pltpu.SemaphoreType.DMA,
            pltpu.SemaphoreType.DMA,
            pltpu.SemaphoreType.DMA,
        ],
    )
    def k(tok_hbm, tab_hbm, pe_hbm, out_hbm,
          idx_v, pe_v, qa, qb, ga, gb, ta, tb, sga, sgb, soa, sob):
        wid = lax.axis_index("s") * NC + lax.axis_index("c")
        j = lax.rem(wid, NSWATH)
        r = wid // NSWATH
        s0 = pl.multiple_of(r * SBLK, 8)
        jb = pl.multiple_of(j * SWATH, SWATH)
        ns = lax.select(r == NRANGE - 1, SMAX, SBLK)

        pltpu.make_async_copy(
            tok_hbm.at[pl.ds(s0, SMAX), pl.ds(jb, SWATH)], idx_v, sga
        ).start()
        pltpu.make_async_copy(
            pe_hbm.at[pl.ds(pl.multiple_of(s0 * EMBED_DIM, 1024),
                            SMAX * EMBED_DIM)], pe_v, sgb
        ).start()
        pltpu.make_async_copy(
            tok_hbm.at[pl.ds(s0, SMAX), pl.ds(jb, SWATH)], idx_v, sga
        ).wait()
        pltpu.make_async_copy(
            pe_hbm.at[pl.ds(pl.multiple_of(s0 * EMBED_DIM, 1024),
                            SMAX * EMBED_DIM)], pe_v, sgb
        ).wait()

        iota = lax.iota(jnp.int32, VEC)
        dcol = [(d * VEC + iota) for d in range(EMBED_DIM // VEC)]

        def make_q(s_local, q):
            for m in range(SWATH // VEC):
                sl = pl.ds(m * VEC, VEC)
                q[sl] = lax.shift_right_logical(idx_v[s_local, sl], 1)

        def gather_start(q, g, sem):
            pltpu.make_async_copy(
                tab_hbm.at[q], g.at[:, pl.ds(0, 2 * EMBED_DIM)], sem).start()

        def gather_wait(q, g, sem):
            pltpu.make_async_copy(
                tab_hbm.at[q], g.at[:, pl.ds(0, 2 * EMBED_DIM)], sem).wait()

        rows = [m * VEC + iota for m in range(SWATH // VEC)]

        def compute(s_local, g, t):
            h64s = [lax.shift_left(idx_v[s_local, pl.ds(m * VEC, VEC)] & 1, 6)
                    for m in range(SWATH // VEC)]

            @plsc.parallel_loop(0, EMBED_DIM, unroll=16)
            def d_body(d):
                pe_spl = plsc.load_gather(
                    pe_v, [jnp.full((VEC,), s_local * EMBED_DIM + d, jnp.int32)])
                for m in range(SWATH // VEC):
                    col = h64s[m] + d
                    v = plsc.load_gather(g, [rows[m], col])
                    t[d, pl.ds(m * VEC, VEC)] = v * 8.0 + pe_spl

        def out_start(s_local, t, sem):
            pltpu.make_async_copy(
                t, out_hbm.at[s0 + s_local, :, pl.ds(jb, SWATH)], sem
            ).start()

        def out_wait(s_local, t, sem):
            pltpu.make_async_copy(
                t, out_hbm.at[s0 + s_local, :, pl.ds(jb, SWATH)], sem
            ).wait()

        make_q(0, qa)
        gather_start(qa, ga, sga)

        def pair_body(ss, carry):
            s_a = 2 * ss
            s_b = 2 * ss + 1
            make_q(s_b, qb)
            gather_start(qb, gb, sgb)
            gather_wait(qa, ga, sga)

            @pl.when(ss > 0)
            def _():
                out_wait(s_a - 2, ta, soa)

            compute(s_a, ga, ta)
            out_start(s_a, ta, soa)

            @pl.when(s_b + 1 < ns)
            def _():
                make_q(s_b + 1, qa)
                gather_start(qa, ga, sga)

            gather_wait(qb, gb, sgb)

            @pl.when(ss > 0)
            def _():
                out_wait(s_b - 2, tb, sob)

            compute(s_b, gb, tb)
            out_start(s_b, tb, sob)
            return carry

        lax.fori_loop(0, ns // 2, pair_body, 0)
        out_wait(ns - 2, ta, soa)
        out_wait(ns - 1, tb, sob)

    return k(tokens_t, table_pairs, pe)


def kernel(tokens, table):
    tokens_t = tokens.T.astype(jnp.int32)
    table_pairs = table.reshape(NUM_VOCAB // 2, 2 * EMBED_DIM)
    out = _sc_embed(tokens_t, table_pairs, jnp.asarray(_PE_FLAT))
    return jnp.transpose(out, (2, 0, 1))
